# Initial kernel scaffold; baseline (speedup 1.0000x reference)
import jax
import jax.numpy as jnp
from jax.experimental import pallas as pl


def kernel(q, k, v, indices):
    raise NotImplementedError("write your pallas kernel here")


# speedup vs baseline: 9.8781x; 9.8781x over previous
import functools
import math

import jax
import jax.numpy as jnp
from jax import lax
from jax.experimental import pallas as pl
from jax.experimental.pallas import tpu as pltpu


def _attn_body(q_ref, k_ref, v_ref, idx_ref, o_ref, *, scale, G, BS, KV, T):
    k = k_ref[0]
    v = v_ref[0]
    idx = idx_ref[0]
    kv_iota = lax.broadcasted_iota(jnp.int32, (BS, KV), 1)
    t_iota = lax.broadcasted_iota(jnp.int32, (BS, T), 1)

    def mb(t, m):
        col = jnp.sum(jnp.where(t_iota == t, idx, 0), axis=1,
                      keepdims=True)
        return m + jnp.where(col == kv_iota, 1.0, 0.0)

    m = lax.fori_loop(0, T, mb, jnp.zeros((BS, KV), jnp.float32))
    valid = m > 0.0
    neg = jnp.float32(-1e30)
    for g in range(G):
        q = q_ref[0, g]
        s = lax.dot_general(q, k, (((1,), (1,)), ((), ())),
                            preferred_element_type=jnp.float32)
        s = jnp.where(valid, s * scale, neg)
        mx = jnp.max(s, axis=1, keepdims=True)
        w = m * jnp.exp(s - mx)
        denom = jnp.sum(w, axis=1, keepdims=True)
        o = lax.dot_general(w, v, (((1,), (0,)), ((), ())),
                            preferred_element_type=jnp.float32)
        o_ref[0, g] = o / denom


def kernel(q, k, v, indices):
    B, Hq, S, D = q.shape
    Hkv = k.shape[1]
    KV = k.shape[2]
    G = Hq // Hkv
    T = indices.shape[-1]
    assert B == 1

    BS = min(256, S)
    qr = q.reshape(Hkv, G, S, D)
    kr = k.reshape(Hkv, KV, D)
    vr = v.reshape(Hkv, KV, D)
    ir = indices.reshape(Hkv, S, T).astype(jnp.int32)

    out = pl.pallas_call(
        functools.partial(_attn_body, scale=1.0 / math.sqrt(D),
                          G=G, BS=BS, KV=KV, T=T),
        grid=(Hkv, S // BS),
        in_specs=[
            pl.BlockSpec((1, G, BS, D), lambda h, s: (h, 0, s, 0)),
            pl.BlockSpec((1, KV, D), lambda h, s: (h, 0, 0)),
            pl.BlockSpec((1, KV, D), lambda h, s: (h, 0, 0)),
            pl.BlockSpec((1, BS, T), lambda h, s: (h, s, 0)),
        ],
        out_specs=pl.BlockSpec((1, G, BS, D), lambda h, s: (h, 0, s, 0)),
        out_shape=jax.ShapeDtypeStruct((Hkv, G, S, D), jnp.float32),
        compiler_params=pltpu.CompilerParams(
            dimension_semantics=("parallel", "parallel")),
    )(qr, kr, vr, ir)
    return out.reshape(B, Hq, S, D)


# speedup vs baseline: 61.1364x; 6.1891x over previous
import functools
import math

import jax
import jax.numpy as jnp
from jax import lax
from jax.experimental import pallas as pl
from jax.experimental.pallas import tpu as pltpu
from jax.experimental.pallas import tpu_sc as plsc


def _make_mbuild(nrows, kv, t, ch):
    info = plsc.get_sparse_core_info()
    nc, ns, nl = info.num_cores, info.num_subcores, info.num_lanes
    nw = nc * ns
    rows_pw = nrows // nw
    nch = rows_pw // ch
    jt = t // nl
    mesh = plsc.VectorSubcoreMesh(core_axis_name="c", subcore_axis_name="s")

    @functools.partial(
        pl.kernel, mesh=mesh,
        out_type=jax.ShapeDtypeStruct((nrows, kv), jnp.float32),
        scratch_types=[
            pltpu.VMEM((ch * jt, nl), jnp.int32),
            pltpu.VMEM((ch, kv), jnp.float32),
        ],
        compiler_params=pltpu.CompilerParams(needs_layout_passes=False),
    )
    def mbuild(idx_hbm, zeros_hbm, m_hbm, idx_v, m_v):
        wid = lax.axis_index("s") * nc + lax.axis_index("c")
        base = wid * rows_pw
        pltpu.sync_copy(zeros_hbm, m_v)
        ones = jnp.full((nl,), 1.0, dtype=jnp.float32)
        zvec = jnp.zeros((nl,), dtype=jnp.float32)

        def chunk_body(c, carry):
            row0 = base + c * ch
            pltpu.sync_copy(idx_hbm.at[pl.ds(row0 * jt, ch * jt)], idx_v)
            for r in range(ch):
                rvec = jnp.full((nl,), r, dtype=jnp.int32)
                for j in range(jt):
                    vals = idx_v[r * jt + j]
                    plsc.addupdate_scatter(m_v, [rvec, vals], ones)
            pltpu.sync_copy(m_v, m_hbm.at[pl.ds(row0, ch)])
            for r in range(ch):
                rvec = jnp.full((nl,), r, dtype=jnp.int32)
                for j in range(jt):
                    vals = idx_v[r * jt + j]
                    plsc.store_scatter(m_v, [rvec, vals], zvec)
            return carry

        lax.fori_loop(0, nch, chunk_body, 0)

    return mbuild


def _attn_body(q_ref, k_ref, v_ref, m_ref, o_ref, *, scale, G):
    k = k_ref[0]
    v = v_ref[0]
    m = m_ref[0]
    valid = m > 0.0
    neg = jnp.float32(-1e30)
    for g in range(G):
        q = q_ref[0, g]
        s = lax.dot_general(q, k, (((1,), (1,)), ((), ())),
                            preferred_element_type=jnp.float32)
        s = jnp.where(valid, s * scale, neg)
        mx = jnp.max(s, axis=1, keepdims=True)
        w = m * jnp.exp(s - mx)
        denom = jnp.sum(w, axis=1, keepdims=True)
        o = lax.dot_general(w, v, (((1,), (0,)), ((), ())),
                            preferred_element_type=jnp.float32)
        o_ref[0, g] = o / denom


def kernel(q, k, v, indices):
    B, Hq, S, D = q.shape
    Hkv = k.shape[1]
    KV = k.shape[2]
    G = Hq // Hkv
    T = indices.shape[-1]
    assert B == 1

    nrows = Hkv * S
    CH = 32
    info = plsc.get_sparse_core_info()
    nl = info.num_lanes

    idx_flat = indices.reshape(nrows * (T // nl), nl).astype(jnp.int32)
    zeros_init = jnp.zeros((CH, KV), jnp.float32)
    m_full = _make_mbuild(nrows, KV, T, CH)(idx_flat, zeros_init)
    m_full = m_full.reshape(Hkv, S, KV)

    BS = min(256, S)
    qr = q.reshape(Hkv, G, S, D)
    kr = k.reshape(Hkv, KV, D)
    vr = v.reshape(Hkv, KV, D)

    out = pl.pallas_call(
        functools.partial(_attn_body, scale=1.0 / math.sqrt(D), G=G),
        grid=(Hkv, S // BS),
        in_specs=[
            pl.BlockSpec((1, G, BS, D), lambda h, s: (h, 0, s, 0)),
            pl.BlockSpec((1, KV, D), lambda h, s: (h, 0, 0)),
            pl.BlockSpec((1, KV, D), lambda h, s: (h, 0, 0)),
            pl.BlockSpec((1, BS, KV), lambda h, s: (h, s, 0)),
        ],
        out_specs=pl.BlockSpec((1, G, BS, D), lambda h, s: (h, 0, s, 0)),
        out_shape=jax.ShapeDtypeStruct((Hkv, G, S, D), jnp.float32),
        compiler_params=pltpu.CompilerParams(
            dimension_semantics=("parallel", "parallel")),
    )(qr, kr, vr, m_full)
    return out.reshape(B, Hq, S, D)


# speedup vs baseline: 61.8805x; 1.0122x over previous
import functools
import math

import jax
import jax.numpy as jnp
from jax import lax
from jax.experimental import pallas as pl
from jax.experimental.pallas import tpu as pltpu
from jax.experimental.pallas import tpu_sc as plsc


def _make_mbuild(nrows, kv, t, ch):
    info = plsc.get_sparse_core_info()
    nc, ns, nl = info.num_cores, info.num_subcores, info.num_lanes
    nw = nc * ns
    rows_pw = nrows // nw
    nch = rows_pw // ch
    jt = t // nl
    mesh = plsc.VectorSubcoreMesh(core_axis_name="c", subcore_axis_name="s")

    @functools.partial(
        pl.kernel, mesh=mesh,
        out_type=jax.ShapeDtypeStruct((nrows, kv), jnp.float32),
        scratch_types=[
            pltpu.VMEM((ch * jt, nl), jnp.int32),
            pltpu.VMEM((ch, kv), jnp.float32),
        ],
        compiler_params=pltpu.CompilerParams(needs_layout_passes=False),
    )
    def mbuild(idx_hbm, zeros_hbm, m_hbm, idx_v, m_v):
        wid = lax.axis_index("s") * nc + lax.axis_index("c")
        base = wid * rows_pw
        pltpu.sync_copy(zeros_hbm, m_v)
        ones = jnp.full((nl,), 1.0, dtype=jnp.float32)
        zvec = jnp.zeros((nl,), dtype=jnp.float32)

        def chunk_body(c, carry):
            row0 = base + c * ch
            pltpu.sync_copy(idx_hbm.at[pl.ds(row0 * jt, ch * jt)], idx_v)
            for r in range(ch):
                rvec = jnp.full((nl,), r, dtype=jnp.int32)
                for j in range(jt):
                    vals = idx_v[r * jt + j]
                    plsc.addupdate_scatter(m_v, [rvec, vals], ones)
            pltpu.sync_copy(m_v, m_hbm.at[pl.ds(row0, ch)])
            for r in range(ch):
                rvec = jnp.full((nl,), r, dtype=jnp.int32)
                for j in range(jt):
                    vals = idx_v[r * jt + j]
                    plsc.store_scatter(m_v, [rvec, vals], zvec)
            return carry

        lax.fori_loop(0, nch, chunk_body, 0)

    return mbuild


def _attn_body(q_ref, k_ref, v_ref, m_ref, o_ref, *, scale, G):
    k = k_ref[0]
    v = v_ref[0]
    m = m_ref[0]
    logm = jnp.log(m)
    for g in range(G):
        q = q_ref[0, g]
        s = lax.dot_general(q, k, (((1,), (1,)), ((), ())),
                            preferred_element_type=jnp.float32)
        s = s * scale + logm
        mx = jnp.max(s, axis=1, keepdims=True)
        w = jnp.exp(s - mx)
        denom = jnp.sum(w, axis=1, keepdims=True)
        o = lax.dot_general(w.astype(jnp.bfloat16), v,
                            (((1,), (0,)), ((), ())),
                            preferred_element_type=jnp.float32)
        o_ref[0, g] = o / denom


def kernel(q, k, v, indices):
    B, Hq, S, D = q.shape
    Hkv = k.shape[1]
    KV = k.shape[2]
    G = Hq // Hkv
    T = indices.shape[-1]
    assert B == 1

    nrows = Hkv * S
    CH = 32
    info = plsc.get_sparse_core_info()
    nl = info.num_lanes

    idx_flat = indices.reshape(nrows * (T // nl), nl).astype(jnp.int32)
    zeros_init = jnp.zeros((CH, KV), jnp.float32)
    m_full = _make_mbuild(nrows, KV, T, CH)(idx_flat, zeros_init)
    m_full = m_full.reshape(Hkv, S, KV)

    BS = min(256, S)
    qr = q.reshape(Hkv, G, S, D).astype(jnp.bfloat16)
    kr = k.reshape(Hkv, KV, D).astype(jnp.bfloat16)
    vr = v.reshape(Hkv, KV, D).astype(jnp.bfloat16)

    out = pl.pallas_call(
        functools.partial(_attn_body, scale=1.0 / math.sqrt(D), G=G),
        grid=(Hkv, S // BS),
        in_specs=[
            pl.BlockSpec((1, G, BS, D), lambda h, s: (h, 0, s, 0)),
            pl.BlockSpec((1, KV, D), lambda h, s: (h, 0, 0)),
            pl.BlockSpec((1, KV, D), lambda h, s: (h, 0, 0)),
            pl.BlockSpec((1, BS, KV), lambda h, s: (h, s, 0)),
        ],
        out_specs=pl.BlockSpec((1, G, BS, D), lambda h, s: (h, 0, s, 0)),
        out_shape=jax.ShapeDtypeStruct((Hkv, G, S, D), jnp.float32),
        compiler_params=pltpu.CompilerParams(
            dimension_semantics=("parallel", "parallel")),
    )(qr, kr, vr, m_full)
    return out.reshape(B, Hq, S, D)


# speedup vs baseline: 64.2210x; 1.0378x over previous
import functools
import math

import jax
import jax.numpy as jnp
from jax import lax
from jax.experimental import pallas as pl
from jax.experimental.pallas import tpu as pltpu
from jax.experimental.pallas import tpu_sc as plsc


def _make_mbuild(nrows, kv, t, ch):
    info = plsc.get_sparse_core_info()
    nc, ns, nl = info.num_cores, info.num_subcores, info.num_lanes
    nw = nc * ns
    rows_pw = nrows // nw
    nch = rows_pw // ch
    jt = t // nl
    mesh = plsc.VectorSubcoreMesh(core_axis_name="c", subcore_axis_name="s")

    @functools.partial(
        pl.kernel, mesh=mesh,
        out_type=jax.ShapeDtypeStruct((nrows, kv), jnp.float32),
        scratch_types=[
            pltpu.VMEM((ch * jt, nl), jnp.int32),
            pltpu.VMEM((ch, kv), jnp.float32),
        ],
        compiler_params=pltpu.CompilerParams(needs_layout_passes=False),
    )
    def mbuild(idx_hbm, zeros_hbm, m_hbm, idx_v, m_v):
        wid = lax.axis_index("s") * nc + lax.axis_index("c")
        base = wid * rows_pw
        pltpu.sync_copy(zeros_hbm, m_v)
        ones = jnp.full((nl,), 1.0, dtype=jnp.float32)
        zvec = jnp.zeros((nl,), dtype=jnp.float32)

        def chunk_body(c, carry):
            row0 = base + c * ch
            pltpu.sync_copy(idx_hbm.at[pl.ds(row0 * jt, ch * jt)], idx_v)
            for r in range(ch):
                rvec = jnp.full((nl,), r, dtype=jnp.int32)
                for j in range(jt):
                    vals = idx_v[r * jt + j]
                    plsc.addupdate_scatter(m_v, [rvec, vals], ones)
            pltpu.sync_copy(m_v, m_hbm.at[pl.ds(row0, ch)])
            for r in range(ch):
                rvec = jnp.full((nl,), r, dtype=jnp.int32)
                for j in range(jt):
                    vals = idx_v[r * jt + j]
                    plsc.store_scatter(m_v, [rvec, vals], zvec)
            return carry

        lax.fori_loop(0, nch, chunk_body, 0)

    return mbuild


def _attn_body(q_ref, k_ref, v_ref, m_ref, o_ref, *, G):
    k = k_ref[0]
    v = v_ref[0]
    m = m_ref[0]
    logm = jnp.log(m)
    for g in range(G):
        q = q_ref[0, g]
        s = lax.dot_general(q, k, (((1,), (1,)), ((), ())),
                            preferred_element_type=jnp.float32)
        s = s + logm
        mx = jnp.max(s, axis=1, keepdims=True)
        w = jnp.exp(s - mx)
        denom = jnp.sum(w, axis=1, keepdims=True)
        o = lax.dot_general(w.astype(jnp.bfloat16), v,
                            (((1,), (0,)), ((), ())),
                            preferred_element_type=jnp.float32)
        o_ref[0, g] = o / denom


def kernel(q, k, v, indices):
    B, Hq, S, D = q.shape
    Hkv = k.shape[1]
    KV = k.shape[2]
    G = Hq // Hkv
    T = indices.shape[-1]
    assert B == 1

    nrows = Hkv * S
    CH = 32
    info = plsc.get_sparse_core_info()
    nl = info.num_lanes

    idx_flat = indices.reshape(nrows * (T // nl), nl).astype(jnp.int32)
    zeros_init = jnp.zeros((CH, KV), jnp.float32)
    m_full = _make_mbuild(nrows, KV, T, CH)(idx_flat, zeros_init)
    m_full = m_full.reshape(Hkv, S, KV)

    BS = min(512, S)
    qr = (q * (1.0 / math.sqrt(D))).reshape(Hkv, G, S, D).astype(jnp.bfloat16)
    kr = k.reshape(Hkv, KV, D).astype(jnp.bfloat16)
    vr = v.reshape(Hkv, KV, D).astype(jnp.bfloat16)

    out = pl.pallas_call(
        functools.partial(_attn_body, G=G),
        grid=(Hkv, S // BS),
        in_specs=[
            pl.BlockSpec((1, G, BS, D), lambda h, s: (h, 0, s, 0)),
            pl.BlockSpec((1, KV, D), lambda h, s: (h, 0, 0)),
            pl.BlockSpec((1, KV, D), lambda h, s: (h, 0, 0)),
            pl.BlockSpec((1, BS, KV), lambda h, s: (h, s, 0)),
        ],
        out_specs=pl.BlockSpec((1, G, BS, D), lambda h, s: (h, 0, s, 0)),
        out_shape=jax.ShapeDtypeStruct((Hkv, G, S, D), jnp.float32),
        compiler_params=pltpu.CompilerParams(
            dimension_semantics=("parallel", "parallel")),
    )(qr, kr, vr, m_full)
    return out.reshape(B, Hq, S, D)


# speedup vs baseline: 65.0724x; 1.0133x over previous
import functools
import math

import jax
import jax.numpy as jnp
from jax import lax
from jax.experimental import pallas as pl
from jax.experimental.pallas import tpu as pltpu
from jax.experimental.pallas import tpu_sc as plsc


def _make_mbuild(nrows, kv, t, ch):
    info = plsc.get_sparse_core_info()
    nc, ns, nl = info.num_cores, info.num_subcores, info.num_lanes
    nw = nc * ns
    rows_pw = nrows // nw
    nch = rows_pw // ch
    assert nch >= 2 and nch % 2 == 0
    jt = t // nl
    mesh = plsc.VectorSubcoreMesh(core_axis_name="c", subcore_axis_name="s")

    @functools.partial(
        pl.kernel, mesh=mesh,
        out_type=jax.ShapeDtypeStruct((nrows, kv), jnp.float32),
        scratch_types=[
            pltpu.VMEM((ch * jt, nl), jnp.int32),
            pltpu.VMEM((ch * jt, nl), jnp.int32),
            pltpu.VMEM((ch, kv), jnp.float32),
            pltpu.VMEM((ch, kv), jnp.float32),
            pltpu.SemaphoreType.DMA,
            pltpu.SemaphoreType.DMA,
        ],
        compiler_params=pltpu.CompilerParams(needs_layout_passes=False),
    )
    def mbuild(idx_hbm, zeros_hbm, m_hbm, idx_v0, idx_v1, m_v0, m_v1,
               sem0, sem1):
        wid = lax.axis_index("s") * nc + lax.axis_index("c")
        base = wid * rows_pw
        idx_v = (idx_v0, idx_v1)
        m_v = (m_v0, m_v1)
        sem = (sem0, sem1)
        ones = jnp.full((nl,), 1.0, dtype=jnp.float32)
        zvec = jnp.zeros((nl,), dtype=jnp.float32)

        def scatter(buf, idxbuf, val):
            for r in range(ch):
                rvec = jnp.full((nl,), r, dtype=jnp.int32)
                for j in range(jt):
                    vals = idxbuf[r * jt + j]
                    if val is None:
                        plsc.addupdate_scatter(buf, [rvec, vals], ones)
                    else:
                        plsc.store_scatter(buf, [rvec, vals], val)

        def load_scatter_start(c, b):
            row0 = base + c * ch
            pltpu.sync_copy(idx_hbm.at[pl.ds(row0 * jt, ch * jt)], idx_v[b])
            scatter(m_v[b], idx_v[b], None)
            pltpu.async_copy(m_v[b], m_hbm.at[pl.ds(row0, ch)], sem[b])

        pltpu.sync_copy(zeros_hbm, m_v0)
        pltpu.sync_copy(zeros_hbm, m_v1)
        for b in range(2):
            load_scatter_start(b, b)

        def pair_body(i, carry):
            for b in range(2):
                c = 2 + i * 2 + b
                row0 = base + c * ch
                pltpu.make_async_copy(
                    m_v[b], m_hbm.at[pl.ds(row0, ch)], sem[b]).wait()
                scatter(m_v[b], idx_v[b], zvec)
                load_scatter_start(c, b)
            return carry

        lax.fori_loop(0, (nch - 2) // 2, pair_body, 0)

        for b in range(2):
            row0 = base + (nch - 2 + b) * ch
            pltpu.make_async_copy(
                m_v[b], m_hbm.at[pl.ds(row0, ch)], sem[b]).wait()

    return mbuild


def _attn_body(q_ref, k_ref, v_ref, m_ref, o_ref, *, G):
    k = k_ref[0]
    v = v_ref[0]
    m = m_ref[0]
    logm = jnp.log(m)
    for g in range(G):
        q = q_ref[0, g]
        s = lax.dot_general(q, k, (((1,), (1,)), ((), ())),
                            preferred_element_type=jnp.float32)
        s = s + logm
        mx = jnp.max(s, axis=1, keepdims=True)
        w = jnp.exp(s - mx)
        denom = jnp.sum(w, axis=1, keepdims=True)
        o = lax.dot_general(w.astype(jnp.bfloat16), v,
                            (((1,), (0,)), ((), ())),
                            preferred_element_type=jnp.float32)
        o_ref[0, g] = o / denom


def kernel(q, k, v, indices):
    B, Hq, S, D = q.shape
    Hkv = k.shape[1]
    KV = k.shape[2]
    G = Hq // Hkv
    T = indices.shape[-1]
    assert B == 1

    nrows = Hkv * S
    CH = 16
    info = plsc.get_sparse_core_info()
    nl = info.num_lanes

    idx_flat = indices.reshape(nrows * (T // nl), nl).astype(jnp.int32)
    zeros_init = jnp.zeros((CH, KV), jnp.float32)
    m_full = _make_mbuild(nrows, KV, T, CH)(idx_flat, zeros_init)
    m_full = m_full.reshape(Hkv, S, KV)

    BS = min(512, S)
    qr = (q * (1.0 / math.sqrt(D))).reshape(Hkv, G, S, D).astype(jnp.bfloat16)
    kr = k.reshape(Hkv, KV, D).astype(jnp.bfloat16)
    vr = v.reshape(Hkv, KV, D).astype(jnp.bfloat16)

    out = pl.pallas_call(
        functools.partial(_attn_body, G=G),
        grid=(Hkv, S // BS),
        in_specs=[
            pl.BlockSpec((1, G, BS, D), lambda h, s: (h, 0, s, 0)),
            pl.BlockSpec((1, KV, D), lambda h, s: (h, 0, 0)),
            pl.BlockSpec((1, KV, D), lambda h, s: (h, 0, 0)),
            pl.BlockSpec((1, BS, KV), lambda h, s: (h, s, 0)),
        ],
        out_specs=pl.BlockSpec((1, G, BS, D), lambda h, s: (h, 0, s, 0)),
        out_shape=jax.ShapeDtypeStruct((Hkv, G, S, D), jnp.float32),
        compiler_params=pltpu.CompilerParams(
            dimension_semantics=("parallel", "parallel")),
    )(qr, kr, vr, m_full)
    return out.reshape(B, Hq, S, D)


# speedup vs baseline: 70.0255x; 1.0761x over previous
import functools
import math

import jax
import jax.numpy as jnp
from jax import lax
from jax.experimental import pallas as pl
from jax.experimental.pallas import tpu as pltpu
from jax.experimental.pallas import tpu_sc as plsc


def _make_mbuild(nrows_p, kv, t, ch, fields):
    info = plsc.get_sparse_core_info()
    nc, ns, nl = info.num_cores, info.num_subcores, info.num_lanes
    nw = nc * ns
    rows_pw = nrows_p // nw
    nch = rows_pw // ch
    assert nch >= 2 and nch % 2 == 0
    jt = t // nl
    mesh = plsc.VectorSubcoreMesh(core_axis_name="c", subcore_axis_name="s")

    @functools.partial(
        pl.kernel, mesh=mesh,
        out_type=jax.ShapeDtypeStruct((nrows_p, kv), jnp.int32),
        scratch_types=[
            pltpu.VMEM((ch * fields * jt, nl), jnp.int32),
            pltpu.VMEM((ch * fields * jt, nl), jnp.int32),
            pltpu.VMEM((ch, kv), jnp.int32),
            pltpu.VMEM((ch, kv), jnp.int32),
            pltpu.SemaphoreType.DMA,
            pltpu.SemaphoreType.DMA,
        ],
        compiler_params=pltpu.CompilerParams(needs_layout_passes=False),
    )
    def mbuild(idx_hbm, zeros_hbm, m_hbm, idx_v0, idx_v1, m_v0, m_v1,
               sem0, sem1):
        wid = lax.axis_index("s") * nc + lax.axis_index("c")
        base = wid * rows_pw
        idx_v = (idx_v0, idx_v1)
        m_v = (m_v0, m_v1)
        sem = (sem0, sem1)
        wvecs = [jnp.full((nl,), 1 << (8 * k), dtype=jnp.int32)
                 for k in range(fields)]
        zvec = jnp.zeros((nl,), dtype=jnp.int32)

        def scatter(buf, idxbuf, zero):
            for r in range(ch):
                rvec = jnp.full((nl,), r, dtype=jnp.int32)
                for k in range(fields):
                    for j in range(jt):
                        vals = idxbuf[(r * fields + k) * jt + j]
                        if zero:
                            plsc.store_scatter(buf, [rvec, vals], zvec)
                        else:
                            plsc.addupdate_scatter(buf, [rvec, vals],
                                                   wvecs[k])

        def load_scatter_start(c, b):
            row0 = base + c * ch
            pltpu.sync_copy(
                idx_hbm.at[pl.ds(row0 * fields * jt, ch * fields * jt)],
                idx_v[b])
            scatter(m_v[b], idx_v[b], False)
            pltpu.async_copy(m_v[b], m_hbm.at[pl.ds(row0, ch)], sem[b])

        pltpu.sync_copy(zeros_hbm, m_v0)
        pltpu.sync_copy(zeros_hbm, m_v1)
        for b in range(2):
            load_scatter_start(b, b)

        def pair_body(i, carry):
            for b in range(2):
                c = 2 + i * 2 + b
                row0 = base + c * ch
                pltpu.make_async_copy(
                    m_v[b], m_hbm.at[pl.ds(row0, ch)], sem[b]).wait()
                scatter(m_v[b], idx_v[b], True)
                load_scatter_start(c, b)
            return carry

        lax.fori_loop(0, (nch - 2) // 2, pair_body, 0)

        for b in range(2):
            row0 = base + (nch - 2 + b) * ch
            pltpu.make_async_copy(
                m_v[b], m_hbm.at[pl.ds(row0, ch)], sem[b]).wait()

    return mbuild


def _attn_body(q_ref, k_ref, v_ref, m_ref, o_ref, *, G):
    k = k_ref[0]
    v = v_ref[0]
    mp = m_ref[0]
    pid = pl.program_id(1)
    cnt = lax.shift_right_logical(mp, pid * 8) & 255
    logm = jnp.log(cnt.astype(jnp.float32))
    for g in range(G):
        q = q_ref[0, g]
        s = lax.dot_general(q, k, (((1,), (1,)), ((), ())),
                            preferred_element_type=jnp.float32)
        s = s + logm
        mx = jnp.max(s, axis=1, keepdims=True)
        w = jnp.exp(s - mx)
        denom = jnp.sum(w, axis=1, keepdims=True)
        o = lax.dot_general(w.astype(jnp.bfloat16), v,
                            (((1,), (0,)), ((), ())),
                            preferred_element_type=jnp.float32)
        o_ref[0, g] = o / denom


def kernel(q, k, v, indices):
    B, Hq, S, D = q.shape
    Hkv = k.shape[1]
    KV = k.shape[2]
    G = Hq // Hkv
    T = indices.shape[-1]
    assert B == 1

    F = 4
    S4 = S // F
    nrows_p = Hkv * S4
    CH = 8
    info = plsc.get_sparse_core_info()
    nl = info.num_lanes

    idx_flat = (indices.reshape(Hkv, F, S4, T).transpose(0, 2, 1, 3)
                .reshape(nrows_p * F * (T // nl), nl).astype(jnp.int32))
    zeros_init = jnp.zeros((CH, KV), jnp.int32)
    m_packed = _make_mbuild(nrows_p, KV, T, CH, F)(idx_flat, zeros_init)
    m_packed = m_packed.reshape(Hkv, S4, KV)

    BS = S4
    qr = (q * (1.0 / math.sqrt(D))).reshape(Hkv, G, S, D).astype(jnp.bfloat16)
    kr = k.reshape(Hkv, KV, D).astype(jnp.bfloat16)
    vr = v.reshape(Hkv, KV, D).astype(jnp.bfloat16)

    out = pl.pallas_call(
        functools.partial(_attn_body, G=G),
        grid=(Hkv, F),
        in_specs=[
            pl.BlockSpec((1, G, BS, D), lambda h, s: (h, 0, s, 0)),
            pl.BlockSpec((1, KV, D), lambda h, s: (h, 0, 0)),
            pl.BlockSpec((1, KV, D), lambda h, s: (h, 0, 0)),
            pl.BlockSpec((1, S4, KV), lambda h, s: (h, 0, 0)),
        ],
        out_specs=pl.BlockSpec((1, G, BS, D), lambda h, s: (h, 0, s, 0)),
        out_shape=jax.ShapeDtypeStruct((Hkv, G, S, D), jnp.float32),
        compiler_params=pltpu.CompilerParams(
            dimension_semantics=("parallel", "parallel")),
    )(qr, kr, vr, m_packed)
    return out.reshape(B, Hq, S, D)
